# Initial kernel scaffold; baseline (speedup 1.0000x reference)
import jax
import jax.numpy as jnp
from jax.experimental import pallas as pl


def kernel(x, edge_index, batch, W_rel1, b_rel1, W_root1, w_pool1, W_rel2, b_rel2, W_root2, w_pool2, W_rel3, b_rel3, W_root3, lin1_w, lin1_b, lin2_w, lin2_b, lin3_w, lin3_b):
    raise NotImplementedError("write your pallas kernel here")


# speedup vs baseline: 6.7203x; 6.7203x over previous
import functools
import math

import jax
import jax.numpy as jnp
from jax import lax
from jax.experimental import pallas as pl
from jax.experimental.pallas import tpu as pltpu
from jax.experimental.pallas import tpu_sc as plsc

N = 10000
E = 320000
G = 50
N0 = 200
FEAT = 128
NEG = -3.0e38

K1 = int(math.ceil(0.8 * N0))
K2 = int(math.ceil(0.8 * K1))
K3 = int(math.ceil(0.8 * K2))

NC = 1
NS = 16
NW = NC * NS
EPW = E // NW
CH = 80
NCH = EPW // CH
RPT = 632
NPAD = RPT * NS


def _make_segsum(D):
    mesh = plsc.VectorSubcoreMesh(core_axis_name="c", subcore_axis_name="s",
                                  num_cores=NC)

    @functools.partial(
        pl.kernel,
        mesh=mesh,
        out_type=jax.ShapeDtypeStruct((NPAD, D), jnp.float32),
        scratch_types=[
            pltpu.VMEM((CH,), jnp.int32),
            pltpu.VMEM((CH,), jnp.int32),
            pltpu.VMEM((CH, D), jnp.float32),
            pltpu.VMEM_SHARED((NPAD, D), jnp.float32),
            pltpu.SemaphoreType.DMA,
        ],
    )
    def seg(h_hbm, src_hbm, dst_hbm, out_hbm, src_v, dst_v, rows_v, acc, sem):
        sid = lax.axis_index("s")
        wid = sid

        def zrow(r, _):
            for j in range(D // 16):
                rows_v[r, pl.ds(16 * j, 16)] = jnp.zeros((16,), jnp.float32)
            return 0

        lax.fori_loop(0, CH, zrow, 0)
        for j in range(RPT // CH):
            pltpu.sync_copy(rows_v, acc.at[pl.ds(sid * RPT + j * CH, CH)])
        rem = RPT % CH
        if rem:
            pltpu.sync_copy(rows_v.at[pl.ds(0, rem)],
                            acc.at[pl.ds(sid * RPT + (RPT // CH) * CH, rem)])
        plsc.subcore_barrier()

        def body(c, _):
            base = wid * EPW + c * CH
            pltpu.sync_copy(src_hbm.at[pl.ds(base, CH)], src_v)
            pltpu.sync_copy(dst_hbm.at[pl.ds(base, CH)], dst_v)
            pltpu.async_copy(h_hbm.at[src_v], rows_v, sem).wait()
            pltpu.sync_copy(rows_v, acc.at[dst_v], add=True)
            return 0

        lax.fori_loop(0, NCH, body, 0)
        plsc.subcore_barrier()

        pltpu.sync_copy(acc.at[pl.ds(sid * RPT, RPT)],
                        out_hbm.at[pl.ds(sid * RPT, RPT)])

    return seg


_segsum128 = _make_segsum(FEAT)


BROWS = 1000


def _dense_body(a_ref, xin_ref, wrt_ref, wot_ref, b_ref, wp_ref, nrm_ref,
                h_ref, s_ref, *, din):
    a = a_ref[:, :din]
    h = jnp.dot(a, wrt_ref[...], preferred_element_type=jnp.float32)
    h = h + jnp.dot(xin_ref[...], wot_ref[...],
                    preferred_element_type=jnp.float32)
    h = jnp.maximum(h + b_ref[...], 0.0)
    h_ref[...] = h
    s = jnp.dot(h, wp_ref[...], preferred_element_type=jnp.float32)
    s_ref[...] = s / nrm_ref[...]


def _dense(A, xin, wrt, wot, b, wp, nrm, din, dw):
    grid = N // BROWS
    return pl.pallas_call(
        functools.partial(_dense_body, din=din),
        grid=(grid,),
        in_specs=[
            pl.BlockSpec((BROWS, dw), lambda i: (i, 0)),
            pl.BlockSpec((BROWS, din), lambda i: (i, 0)),
            pl.BlockSpec((din, FEAT), lambda i: (0, 0)),
            pl.BlockSpec((din, FEAT), lambda i: (0, 0)),
            pl.BlockSpec((1, FEAT), lambda i: (0, 0)),
            pl.BlockSpec((FEAT, 1), lambda i: (0, 0)),
            pl.BlockSpec((1, 1), lambda i: (0, 0)),
        ],
        out_specs=[
            pl.BlockSpec((BROWS, FEAT), lambda i: (i, 0)),
            pl.BlockSpec((BROWS, 1), lambda i: (i, 0)),
        ],
        out_shape=[
            jax.ShapeDtypeStruct((N, FEAT), jnp.float32),
            jax.ShapeDtypeStruct((N, 1), jnp.float32),
        ],
    )(A, xin, wrt, wot, b, wp, nrm)


def _select_body(h_ref, sc_ref, sr_ref, tc_ref, tr_ref,
                 H_ref, mask_ref, rank_ref, xmax_ref, xsum_ref, *, k):
    scol = sc_ref[...]
    srow = sr_ref[0]
    tcol = tc_ref[...]
    trow = tr_ref[0]
    si = jnp.broadcast_to(scol, (N0, N0))
    sj = jnp.broadcast_to(srow, (N0, N0))
    ti = jnp.broadcast_to(tcol, (N0, N0))
    tj = jnp.broadcast_to(trow, (N0, N0))
    cmp = (sj > si) | ((sj == si) & (tj < ti))
    rank = jnp.sum(cmp.astype(jnp.float32), axis=1, keepdims=True)
    mask = rank < k
    gate = jnp.where(mask, scol, 0.0)
    Hb = h_ref[...] * gate
    H_ref[...] = Hb
    mask_ref[...] = mask.astype(jnp.float32)
    rank_ref[...] = rank
    xmax_ref[0] = jnp.max(jnp.where(mask, Hb, NEG), axis=0, keepdims=True)
    xsum_ref[0] = jnp.sum(Hb, axis=0, keepdims=True) / float(k)


def _select(h, s_col, tk_col, k):
    H, mask, rank, xmax, xsum = pl.pallas_call(
        functools.partial(_select_body, k=k),
        grid=(G,),
        in_specs=[
            pl.BlockSpec((N0, FEAT), lambda g: (g, 0)),
            pl.BlockSpec((N0, 1), lambda g: (g, 0)),
            pl.BlockSpec((1, 1, N0), lambda g: (g, 0, 0)),
            pl.BlockSpec((N0, 1), lambda g: (g, 0)),
            pl.BlockSpec((1, 1, N0), lambda g: (g, 0, 0)),
        ],
        out_specs=[
            pl.BlockSpec((N0, FEAT), lambda g: (g, 0)),
            pl.BlockSpec((N0, 1), lambda g: (g, 0)),
            pl.BlockSpec((N0, 1), lambda g: (g, 0)),
            pl.BlockSpec((1, 1, FEAT), lambda g: (g, 0, 0)),
            pl.BlockSpec((1, 1, FEAT), lambda g: (g, 0, 0)),
        ],
        out_shape=[
            jax.ShapeDtypeStruct((N, FEAT), jnp.float32),
            jax.ShapeDtypeStruct((N, 1), jnp.float32),
            jax.ShapeDtypeStruct((N, 1), jnp.float32),
            jax.ShapeDtypeStruct((G, 1, FEAT), jnp.float32),
            jax.ShapeDtypeStruct((G, 1, FEAT), jnp.float32),
        ],
    )(h, s_col, s_col.reshape(G, 1, N0), tk_col, tk_col.reshape(G, 1, N0))
    return H, mask, rank, xmax.reshape(G, FEAT), xsum.reshape(G, FEAT)


def _head_body(x1m, x2m, x3m, x1s, x2s, x3s, w1a, w1b, b1, w2, b2, w3, b3,
               out_ref):
    zm = x1m[...] + x2m[...] + x3m[...]
    zs = x1s[...] + x2s[...] + x3s[...]
    z = jnp.dot(zm, w1a[...], preferred_element_type=jnp.float32)
    z = z + jnp.dot(zs, w1b[...], preferred_element_type=jnp.float32)
    z = jnp.maximum(z + b1[...], 0.0)
    z = jnp.maximum(jnp.dot(z, w2[...], preferred_element_type=jnp.float32)
                    + b2[...], 0.0)
    o = jnp.dot(z, w3[...], preferred_element_type=jnp.float32) + b3[...]
    t = o - jnp.max(o, axis=1, keepdims=True)
    out_ref[...] = t - jnp.log(jnp.sum(jnp.exp(t), axis=1, keepdims=True))


def _head(x1m, x2m, x3m, x1s, x2s, x3s, w1a, w1b, b1, w2, b2, w3, b3):
    return pl.pallas_call(
        _head_body,
        out_shape=jax.ShapeDtypeStruct((G, 7), jnp.float32),
    )(x1m, x2m, x3m, x1s, x2s, x3s, w1a, w1b, b1, w2, b2, w3, b3)


def kernel(x, edge_index, batch, W_rel1, b_rel1, W_root1, w_pool1, W_rel2,
           b_rel2, W_root2, w_pool2, W_rel3, b_rel3, W_root3, lin1_w, lin1_b,
           lin2_w, lin2_b, lin3_w, lin3_b):
    src = edge_index[0]
    dst = edge_index[1]

    x128 = jnp.concatenate([x, jnp.zeros((N, FEAT - 4), jnp.float32)], axis=1)
    n1 = jnp.linalg.norm(w_pool1).reshape(1, 1)
    n2 = jnp.linalg.norm(w_pool2).reshape(1, 1)
    w1c = w_pool1.reshape(FEAT, 1)
    w2c = w_pool2.reshape(FEAT, 1)
    tk0 = (jnp.arange(N, dtype=jnp.float32) % N0).reshape(N, 1)

    A1 = _segsum128(x128, src, dst)
    h1, u1 = _dense(A1, x, W_rel1.T, W_root1.T, b_rel1.reshape(1, FEAT),
                    w1c, n1, 4, FEAT)
    s1 = jnp.tanh(u1)
    H1, m1, r1, x1m, x1s = _select(h1, s1, tk0, K1)

    A2 = _segsum128(H1, src, dst)
    h2, u2 = _dense(A2, H1, W_rel2.T, W_root2.T, b_rel2.reshape(1, FEAT),
                    w2c, n2, FEAT, FEAT)
    s2 = jnp.where(m1 > 0, jnp.tanh(u2), NEG)
    H2, m2, r2, x2m, x2s = _select(h2, s2, r1, K2)

    A3 = _segsum128(H2, src, dst)
    h3, u3 = _dense(A3, H2, W_rel3.T, W_root3.T, b_rel3.reshape(1, FEAT),
                    w2c, n2, FEAT, FEAT)
    s3 = jnp.where(m2 > 0, jnp.tanh(u3), NEG)
    _, _, _, x3m, x3s = _select(h3, s3, r2, K3)

    l1t = lin1_w.T
    return _head(x1m, x2m, x3m, x1s, x2s, x3s,
                 l1t[:FEAT], l1t[FEAT:], lin1_b.reshape(1, FEAT),
                 lin2_w.T, lin2_b.reshape(1, 64),
                 lin3_w.T, lin3_b.reshape(1, 7))


# speedup vs baseline: 13.0632x; 1.9438x over previous
import functools
import math

import jax
import jax.numpy as jnp
from jax import lax
from jax.experimental import pallas as pl
from jax.experimental.pallas import tpu as pltpu
from jax.experimental.pallas import tpu_sc as plsc

N = 10000
E = 320000
G = 50
N0 = 200
FEAT = 128
NEG = -3.0e38

K1 = int(math.ceil(0.8 * N0))
K2 = int(math.ceil(0.8 * K1))
K3 = int(math.ceil(0.8 * K2))

NC = 1
NS = 16
NW = NC * NS
EPW = E // NW
CH = 128
NCHF = EPW // CH
TAIL = EPW - NCHF * CH
UNR = 12
RPT = 632
NPAD = RPT * NS


def _make_segsum(D):
    mesh = plsc.VectorSubcoreMesh(core_axis_name="c", subcore_axis_name="s",
                                  num_cores=NC)

    @functools.partial(
        pl.kernel,
        mesh=mesh,
        out_type=jax.ShapeDtypeStruct((NPAD, D), jnp.float32),
        scratch_types=[
            pltpu.VMEM((CH,), jnp.int32),
            pltpu.VMEM((CH,), jnp.int32),
            pltpu.VMEM((CH,), jnp.int32),
            pltpu.VMEM((CH,), jnp.int32),
            pltpu.VMEM((TAIL,), jnp.int32),
            pltpu.VMEM((TAIL,), jnp.int32),
            pltpu.VMEM((CH, D), jnp.float32),
            pltpu.VMEM((CH, D), jnp.float32),
            pltpu.VMEM_SHARED((NPAD, D), jnp.float32),
            pltpu.SemaphoreType.DMA,
            pltpu.SemaphoreType.DMA,
        ],
    )
    def seg(h_hbm, src_hbm, dst_hbm, out_hbm, src0, src1, dst0, dst1,
            srct, dstt, rows0, rows1, acc, sem0, sem1):
        sid = lax.axis_index("s")
        wid = sid
        srcv = (src0, src1)
        dstv = (dst0, dst1)
        rows = (rows0, rows1)
        sems = (sem0, sem1)

        def zrow(r, _):
            for j in range(D // 16):
                rows0[r, pl.ds(16 * j, 16)] = jnp.zeros((16,), jnp.float32)
            return 0

        lax.fori_loop(0, CH, zrow, 0)
        for j in range(RPT // CH):
            pltpu.sync_copy(rows0, acc.at[pl.ds(sid * RPT + j * CH, CH)])
        rem = RPT % CH
        if rem:
            pltpu.sync_copy(rows0.at[pl.ds(0, rem)],
                            acc.at[pl.ds(sid * RPT + (RPT // CH) * CH, rem)])
        plsc.subcore_barrier()

        def start_gather(j, p):
            base = wid * EPW + j * CH
            pltpu.sync_copy(src_hbm.at[pl.ds(base, CH)], srcv[p])
            pltpu.sync_copy(dst_hbm.at[pl.ds(base, CH)], dstv[p])
            pltpu.make_async_copy(h_hbm.at[srcv[p]], rows[p], sems[p]).start()

        start_gather(0, 0)

        def outer(o, _):
            for t in range(UNR):
                p = t % 2
                q = 1 - p
                j = o * UNR + t
                jn = j + 1

                @pl.when(jn < NCHF)
                def _():
                    start_gather(jn, q)

                pltpu.make_async_copy(h_hbm.at[srcv[p]], rows[p],
                                      sems[p]).wait()
                pltpu.sync_copy(rows[p], acc.at[dstv[p]], add=True)
            return 0

        lax.fori_loop(0, NCHF // UNR, outer, 0)

        if TAIL:
            base = wid * EPW + NCHF * CH
            pltpu.sync_copy(src_hbm.at[pl.ds(base, TAIL)], srct)
            pltpu.sync_copy(dst_hbm.at[pl.ds(base, TAIL)], dstt)
            pltpu.async_copy(h_hbm.at[srct], rows0.at[pl.ds(0, TAIL)],
                             sem0).wait()
            pltpu.sync_copy(rows0.at[pl.ds(0, TAIL)], acc.at[dstt], add=True)
        plsc.subcore_barrier()

        pltpu.sync_copy(acc.at[pl.ds(sid * RPT, RPT)],
                        out_hbm.at[pl.ds(sid * RPT, RPT)])

    return seg


_segsum128 = _make_segsum(FEAT)


BROWS = 1000


def _dense_body(a_ref, xin_ref, wrt_ref, wot_ref, b_ref, wp_ref, nrm_ref,
                h_ref, s_ref, *, din):
    a = a_ref[:, :din]
    h = jnp.dot(a, wrt_ref[...], preferred_element_type=jnp.float32)
    h = h + jnp.dot(xin_ref[...], wot_ref[...],
                    preferred_element_type=jnp.float32)
    h = jnp.maximum(h + b_ref[...], 0.0)
    h_ref[...] = h
    s = jnp.dot(h, wp_ref[...], preferred_element_type=jnp.float32)
    s_ref[...] = s / nrm_ref[...]


def _dense(A, xin, wrt, wot, b, wp, nrm, din, dw):
    grid = N // BROWS
    return pl.pallas_call(
        functools.partial(_dense_body, din=din),
        grid=(grid,),
        in_specs=[
            pl.BlockSpec((BROWS, dw), lambda i: (i, 0)),
            pl.BlockSpec((BROWS, din), lambda i: (i, 0)),
            pl.BlockSpec((din, FEAT), lambda i: (0, 0)),
            pl.BlockSpec((din, FEAT), lambda i: (0, 0)),
            pl.BlockSpec((1, FEAT), lambda i: (0, 0)),
            pl.BlockSpec((FEAT, 1), lambda i: (0, 0)),
            pl.BlockSpec((1, 1), lambda i: (0, 0)),
        ],
        out_specs=[
            pl.BlockSpec((BROWS, FEAT), lambda i: (i, 0)),
            pl.BlockSpec((BROWS, 1), lambda i: (i, 0)),
        ],
        out_shape=[
            jax.ShapeDtypeStruct((N, FEAT), jnp.float32),
            jax.ShapeDtypeStruct((N, 1), jnp.float32),
        ],
    )(A, xin, wrt, wot, b, wp, nrm)


def _select_body(h_ref, sc_ref, sr_ref, tc_ref, tr_ref,
                 H_ref, mask_ref, rank_ref, xmax_ref, xsum_ref, *, k):
    scol = sc_ref[...]
    srow = sr_ref[0]
    tcol = tc_ref[...]
    trow = tr_ref[0]
    si = jnp.broadcast_to(scol, (N0, N0))
    sj = jnp.broadcast_to(srow, (N0, N0))
    ti = jnp.broadcast_to(tcol, (N0, N0))
    tj = jnp.broadcast_to(trow, (N0, N0))
    cmp = (sj > si) | ((sj == si) & (tj < ti))
    rank = jnp.sum(cmp.astype(jnp.float32), axis=1, keepdims=True)
    mask = rank < k
    gate = jnp.where(mask, scol, 0.0)
    Hb = h_ref[...] * gate
    H_ref[...] = Hb
    mask_ref[...] = mask.astype(jnp.float32)
    rank_ref[...] = rank
    xmax_ref[0] = jnp.max(jnp.where(mask, Hb, NEG), axis=0, keepdims=True)
    xsum_ref[0] = jnp.sum(Hb, axis=0, keepdims=True) / float(k)


def _select(h, s_col, tk_col, k):
    H, mask, rank, xmax, xsum = pl.pallas_call(
        functools.partial(_select_body, k=k),
        grid=(G,),
        in_specs=[
            pl.BlockSpec((N0, FEAT), lambda g: (g, 0)),
            pl.BlockSpec((N0, 1), lambda g: (g, 0)),
            pl.BlockSpec((1, 1, N0), lambda g: (g, 0, 0)),
            pl.BlockSpec((N0, 1), lambda g: (g, 0)),
            pl.BlockSpec((1, 1, N0), lambda g: (g, 0, 0)),
        ],
        out_specs=[
            pl.BlockSpec((N0, FEAT), lambda g: (g, 0)),
            pl.BlockSpec((N0, 1), lambda g: (g, 0)),
            pl.BlockSpec((N0, 1), lambda g: (g, 0)),
            pl.BlockSpec((1, 1, FEAT), lambda g: (g, 0, 0)),
            pl.BlockSpec((1, 1, FEAT), lambda g: (g, 0, 0)),
        ],
        out_shape=[
            jax.ShapeDtypeStruct((N, FEAT), jnp.float32),
            jax.ShapeDtypeStruct((N, 1), jnp.float32),
            jax.ShapeDtypeStruct((N, 1), jnp.float32),
            jax.ShapeDtypeStruct((G, 1, FEAT), jnp.float32),
            jax.ShapeDtypeStruct((G, 1, FEAT), jnp.float32),
        ],
    )(h, s_col, s_col.reshape(G, 1, N0), tk_col, tk_col.reshape(G, 1, N0))
    return H, mask, rank, xmax.reshape(G, FEAT), xsum.reshape(G, FEAT)


def _head_body(x1m, x2m, x3m, x1s, x2s, x3s, w1a, w1b, b1, w2, b2, w3, b3,
               out_ref):
    zm = x1m[...] + x2m[...] + x3m[...]
    zs = x1s[...] + x2s[...] + x3s[...]
    z = jnp.dot(zm, w1a[...], preferred_element_type=jnp.float32)
    z = z + jnp.dot(zs, w1b[...], preferred_element_type=jnp.float32)
    z = jnp.maximum(z + b1[...], 0.0)
    z = jnp.maximum(jnp.dot(z, w2[...], preferred_element_type=jnp.float32)
                    + b2[...], 0.0)
    o = jnp.dot(z, w3[...], preferred_element_type=jnp.float32) + b3[...]
    t = o - jnp.max(o, axis=1, keepdims=True)
    out_ref[...] = t - jnp.log(jnp.sum(jnp.exp(t), axis=1, keepdims=True))


def _head(x1m, x2m, x3m, x1s, x2s, x3s, w1a, w1b, b1, w2, b2, w3, b3):
    return pl.pallas_call(
        _head_body,
        out_shape=jax.ShapeDtypeStruct((G, 7), jnp.float32),
    )(x1m, x2m, x3m, x1s, x2s, x3s, w1a, w1b, b1, w2, b2, w3, b3)


def kernel(x, edge_index, batch, W_rel1, b_rel1, W_root1, w_pool1, W_rel2,
           b_rel2, W_root2, w_pool2, W_rel3, b_rel3, W_root3, lin1_w, lin1_b,
           lin2_w, lin2_b, lin3_w, lin3_b):
    src = edge_index[0]
    dst = edge_index[1]

    x128 = jnp.concatenate([x, jnp.zeros((N, FEAT - 4), jnp.float32)], axis=1)
    n1 = jnp.linalg.norm(w_pool1).reshape(1, 1)
    n2 = jnp.linalg.norm(w_pool2).reshape(1, 1)
    w1c = w_pool1.reshape(FEAT, 1)
    w2c = w_pool2.reshape(FEAT, 1)
    tk0 = (jnp.arange(N, dtype=jnp.float32) % N0).reshape(N, 1)

    A1 = _segsum128(x128, src, dst)
    h1, u1 = _dense(A1, x, W_rel1.T, W_root1.T, b_rel1.reshape(1, FEAT),
                    w1c, n1, 4, FEAT)
    s1 = jnp.tanh(u1)
    H1, m1, r1, x1m, x1s = _select(h1, s1, tk0, K1)

    A2 = _segsum128(H1, src, dst)
    h2, u2 = _dense(A2, H1, W_rel2.T, W_root2.T, b_rel2.reshape(1, FEAT),
                    w2c, n2, FEAT, FEAT)
    s2 = jnp.where(m1 > 0, jnp.tanh(u2), NEG)
    H2, m2, r2, x2m, x2s = _select(h2, s2, r1, K2)

    A3 = _segsum128(H2, src, dst)
    h3, u3 = _dense(A3, H2, W_rel3.T, W_root3.T, b_rel3.reshape(1, FEAT),
                    w2c, n2, FEAT, FEAT)
    s3 = jnp.where(m2 > 0, jnp.tanh(u3), NEG)
    _, _, _, x3m, x3s = _select(h3, s3, r2, K3)

    l1t = lin1_w.T
    return _head(x1m, x2m, x3m, x1s, x2s, x3s,
                 l1t[:FEAT], l1t[FEAT:], lin1_b.reshape(1, FEAT),
                 lin2_w.T, lin2_b.reshape(1, 64),
                 lin3_w.T, lin3_b.reshape(1, 7))


# speedup vs baseline: 17.2289x; 1.3189x over previous
import functools
import math

import jax
import jax.numpy as jnp
from jax import lax
from jax.experimental import pallas as pl
from jax.experimental.pallas import tpu as pltpu
from jax.experimental.pallas import tpu_sc as plsc

N = 10000
E = 320000
G = 50
N0 = 200
FEAT = 128
NEG = -3.0e38

K1 = int(math.ceil(0.8 * N0))
K2 = int(math.ceil(0.8 * K1))
K3 = int(math.ceil(0.8 * K2))

NC = 1
NS = 16
NW = NC * NS
EPW = E // NW
CH = 128
NCHF = EPW // CH
TAIL = EPW - NCHF * CH
UNR = 12
RPT = 632
NPAD = RPT * NS


def _make_segsum(D):
    mesh = plsc.VectorSubcoreMesh(core_axis_name="c", subcore_axis_name="s",
                                  num_cores=NC)

    @functools.partial(
        pl.kernel,
        mesh=mesh,
        out_type=jax.ShapeDtypeStruct((NPAD, D), jnp.float32),
        scratch_types=[
            pltpu.VMEM((4, CH), jnp.int32),
            pltpu.VMEM((4, CH), jnp.int32),
            pltpu.VMEM((TAIL,), jnp.int32),
            pltpu.VMEM((TAIL,), jnp.int32),
            pltpu.VMEM((CH, D), jnp.float32),
            pltpu.VMEM((CH, D), jnp.float32),
            pltpu.VMEM_SHARED((NPAD, D), jnp.float32),
            pltpu.SemaphoreType.DMA,
            pltpu.SemaphoreType.DMA,
            pltpu.SemaphoreType.DMA,
            pltpu.SemaphoreType.DMA,
            pltpu.SemaphoreType.DMA,
            pltpu.SemaphoreType.DMA,
        ],
    )
    def seg(h_hbm, src_hbm, dst_hbm, out_hbm, srcr, dstr,
            srct, dstt, rows0, rows1, acc,
            gsem0, gsem1, isem0, isem1, isem2, isem3):
        sid = lax.axis_index("s")
        wid = sid
        rows = (rows0, rows1)
        gsems = (gsem0, gsem1)
        isems = (isem0, isem1, isem2, isem3)

        def zrow(r, _):
            for j in range(D // 16):
                rows0[r, pl.ds(16 * j, 16)] = jnp.zeros((16,), jnp.float32)
            return 0

        lax.fori_loop(0, CH, zrow, 0)
        for j in range(RPT // CH):
            pltpu.sync_copy(rows0, acc.at[pl.ds(sid * RPT + j * CH, CH)])
        rem = RPT % CH
        if rem:
            pltpu.sync_copy(rows0.at[pl.ds(0, rem)],
                            acc.at[pl.ds(sid * RPT + (RPT // CH) * CH, rem)])
        plsc.subcore_barrier()

        def idx_copies(j, r):
            base = wid * EPW + j * CH
            return (pltpu.make_async_copy(src_hbm.at[pl.ds(base, CH)],
                                          srcr.at[r], isems[r]),
                    pltpu.make_async_copy(dst_hbm.at[pl.ds(base, CH)],
                                          dstr.at[r], isems[r]))

        def start_idx(j, r):
            a, b = idx_copies(j, r)
            a.start()
            b.start()

        def wait_idx(j, r):
            a, b = idx_copies(j, r)
            a.wait()
            b.wait()

        def gather_copy(r, p):
            return pltpu.make_async_copy(h_hbm.at[srcr.at[r]], rows[p],
                                         gsems[p])

        start_idx(0, 0)
        start_idx(1, 1)
        wait_idx(0, 0)
        gather_copy(0, 0).start()

        def outer(o, _):
            for t in range(UNR):
                p = t % 2
                r = t % 4
                j = o * UNR + t

                @pl.when(j + 2 < NCHF)
                def _():
                    start_idx(j + 2, (r + 2) % 4)

                @pl.when(j + 1 < NCHF)
                def _():
                    wait_idx(j + 1, (r + 1) % 4)
                    gather_copy((r + 1) % 4, 1 - p).start()

                gather_copy(r, p).wait()
                pltpu.sync_copy(rows[p], acc.at[dstr.at[r]], add=True)
            return 0

        lax.fori_loop(0, NCHF // UNR, outer, 0)

        if TAIL:
            base = wid * EPW + NCHF * CH
            pltpu.sync_copy(src_hbm.at[pl.ds(base, TAIL)], srct)
            pltpu.sync_copy(dst_hbm.at[pl.ds(base, TAIL)], dstt)
            pltpu.async_copy(h_hbm.at[srct], rows0.at[pl.ds(0, TAIL)],
                             gsem0).wait()
            pltpu.sync_copy(rows0.at[pl.ds(0, TAIL)], acc.at[dstt], add=True)
        plsc.subcore_barrier()

        pltpu.sync_copy(acc.at[pl.ds(sid * RPT, RPT)],
                        out_hbm.at[pl.ds(sid * RPT, RPT)])

    return seg


_segsum128 = _make_segsum(FEAT)


BROWS = 1000


def _dense_body(a_ref, xin_ref, wrt_ref, wot_ref, b_ref, wp_ref, nrm_ref,
                h_ref, s_ref, *, din):
    a = a_ref[:, :din]
    h = jnp.dot(a, wrt_ref[...], preferred_element_type=jnp.float32)
    h = h + jnp.dot(xin_ref[...], wot_ref[...],
                    preferred_element_type=jnp.float32)
    h = jnp.maximum(h + b_ref[...], 0.0)
    h_ref[...] = h
    s = jnp.dot(h, wp_ref[...], preferred_element_type=jnp.float32)
    s_ref[...] = s / nrm_ref[...]


def _dense(A, xin, wrt, wot, b, wp, nrm, din, dw):
    grid = N // BROWS
    return pl.pallas_call(
        functools.partial(_dense_body, din=din),
        grid=(grid,),
        in_specs=[
            pl.BlockSpec((BROWS, dw), lambda i: (i, 0)),
            pl.BlockSpec((BROWS, din), lambda i: (i, 0)),
            pl.BlockSpec((din, FEAT), lambda i: (0, 0)),
            pl.BlockSpec((din, FEAT), lambda i: (0, 0)),
            pl.BlockSpec((1, FEAT), lambda i: (0, 0)),
            pl.BlockSpec((FEAT, 1), lambda i: (0, 0)),
            pl.BlockSpec((1, 1), lambda i: (0, 0)),
        ],
        out_specs=[
            pl.BlockSpec((BROWS, FEAT), lambda i: (i, 0)),
            pl.BlockSpec((BROWS, 1), lambda i: (i, 0)),
        ],
        out_shape=[
            jax.ShapeDtypeStruct((N, FEAT), jnp.float32),
            jax.ShapeDtypeStruct((N, 1), jnp.float32),
        ],
    )(A, xin, wrt, wot, b, wp, nrm)


def _select_body(h_ref, sc_ref, sr_ref, tc_ref, tr_ref,
                 H_ref, mask_ref, rank_ref, xmax_ref, xsum_ref, *, k):
    scol = sc_ref[...]
    srow = sr_ref[0]
    tcol = tc_ref[...]
    trow = tr_ref[0]
    si = jnp.broadcast_to(scol, (N0, N0))
    sj = jnp.broadcast_to(srow, (N0, N0))
    ti = jnp.broadcast_to(tcol, (N0, N0))
    tj = jnp.broadcast_to(trow, (N0, N0))
    cmp = (sj > si) | ((sj == si) & (tj < ti))
    rank = jnp.sum(cmp.astype(jnp.float32), axis=1, keepdims=True)
    mask = rank < k
    gate = jnp.where(mask, scol, 0.0)
    Hb = h_ref[...] * gate
    H_ref[...] = Hb
    mask_ref[...] = mask.astype(jnp.float32)
    rank_ref[...] = rank
    xmax_ref[0] = jnp.max(jnp.where(mask, Hb, NEG), axis=0, keepdims=True)
    xsum_ref[0] = jnp.sum(Hb, axis=0, keepdims=True) / float(k)


def _select(h, s_col, tk_col, k):
    H, mask, rank, xmax, xsum = pl.pallas_call(
        functools.partial(_select_body, k=k),
        grid=(G,),
        in_specs=[
            pl.BlockSpec((N0, FEAT), lambda g: (g, 0)),
            pl.BlockSpec((N0, 1), lambda g: (g, 0)),
            pl.BlockSpec((1, 1, N0), lambda g: (g, 0, 0)),
            pl.BlockSpec((N0, 1), lambda g: (g, 0)),
            pl.BlockSpec((1, 1, N0), lambda g: (g, 0, 0)),
        ],
        out_specs=[
            pl.BlockSpec((N0, FEAT), lambda g: (g, 0)),
            pl.BlockSpec((N0, 1), lambda g: (g, 0)),
            pl.BlockSpec((N0, 1), lambda g: (g, 0)),
            pl.BlockSpec((1, 1, FEAT), lambda g: (g, 0, 0)),
            pl.BlockSpec((1, 1, FEAT), lambda g: (g, 0, 0)),
        ],
        out_shape=[
            jax.ShapeDtypeStruct((N, FEAT), jnp.float32),
            jax.ShapeDtypeStruct((N, 1), jnp.float32),
            jax.ShapeDtypeStruct((N, 1), jnp.float32),
            jax.ShapeDtypeStruct((G, 1, FEAT), jnp.float32),
            jax.ShapeDtypeStruct((G, 1, FEAT), jnp.float32),
        ],
    )(h, s_col, s_col.reshape(G, 1, N0), tk_col, tk_col.reshape(G, 1, N0))
    return H, mask, rank, xmax.reshape(G, FEAT), xsum.reshape(G, FEAT)


def _head_body(x1m, x2m, x3m, x1s, x2s, x3s, w1a, w1b, b1, w2, b2, w3, b3,
               out_ref):
    zm = x1m[...] + x2m[...] + x3m[...]
    zs = x1s[...] + x2s[...] + x3s[...]
    z = jnp.dot(zm, w1a[...], preferred_element_type=jnp.float32)
    z = z + jnp.dot(zs, w1b[...], preferred_element_type=jnp.float32)
    z = jnp.maximum(z + b1[...], 0.0)
    z = jnp.maximum(jnp.dot(z, w2[...], preferred_element_type=jnp.float32)
                    + b2[...], 0.0)
    o = jnp.dot(z, w3[...], preferred_element_type=jnp.float32) + b3[...]
    t = o - jnp.max(o, axis=1, keepdims=True)
    out_ref[...] = t - jnp.log(jnp.sum(jnp.exp(t), axis=1, keepdims=True))


def _head(x1m, x2m, x3m, x1s, x2s, x3s, w1a, w1b, b1, w2, b2, w3, b3):
    return pl.pallas_call(
        _head_body,
        out_shape=jax.ShapeDtypeStruct((G, 7), jnp.float32),
    )(x1m, x2m, x3m, x1s, x2s, x3s, w1a, w1b, b1, w2, b2, w3, b3)


def kernel(x, edge_index, batch, W_rel1, b_rel1, W_root1, w_pool1, W_rel2,
           b_rel2, W_root2, w_pool2, W_rel3, b_rel3, W_root3, lin1_w, lin1_b,
           lin2_w, lin2_b, lin3_w, lin3_b):
    src = edge_index[0]
    dst = edge_index[1]

    x128 = jnp.concatenate([x, jnp.zeros((N, FEAT - 4), jnp.float32)], axis=1)
    n1 = jnp.linalg.norm(w_pool1).reshape(1, 1)
    n2 = jnp.linalg.norm(w_pool2).reshape(1, 1)
    w1c = w_pool1.reshape(FEAT, 1)
    w2c = w_pool2.reshape(FEAT, 1)
    tk0 = (jnp.arange(N, dtype=jnp.float32) % N0).reshape(N, 1)

    A1 = _segsum128(x128, src, dst)
    h1, u1 = _dense(A1, x, W_rel1.T, W_root1.T, b_rel1.reshape(1, FEAT),
                    w1c, n1, 4, FEAT)
    s1 = jnp.tanh(u1)
    H1, m1, r1, x1m, x1s = _select(h1, s1, tk0, K1)

    A2 = _segsum128(H1, src, dst)
    h2, u2 = _dense(A2, H1, W_rel2.T, W_root2.T, b_rel2.reshape(1, FEAT),
                    w2c, n2, FEAT, FEAT)
    s2 = jnp.where(m1 > 0, jnp.tanh(u2), NEG)
    H2, m2, r2, x2m, x2s = _select(h2, s2, r1, K2)

    A3 = _segsum128(H2, src, dst)
    h3, u3 = _dense(A3, H2, W_rel3.T, W_root3.T, b_rel3.reshape(1, FEAT),
                    w2c, n2, FEAT, FEAT)
    s3 = jnp.where(m2 > 0, jnp.tanh(u3), NEG)
    _, _, _, x3m, x3s = _select(h3, s3, r2, K3)

    l1t = lin1_w.T
    return _head(x1m, x2m, x3m, x1s, x2s, x3s,
                 l1t[:FEAT], l1t[FEAT:], lin1_b.reshape(1, FEAT),
                 lin2_w.T, lin2_b.reshape(1, 64),
                 lin3_w.T, lin3_b.reshape(1, 7))
